# Initial kernel scaffold; baseline (speedup 1.0000x reference)
import functools

import jax
import jax.numpy as jnp
import numpy as np
from jax import lax
from jax.experimental import pallas as pl
from jax.experimental.pallas import tpu as pltpu
from jax.experimental.pallas import tpu_sc as plsc

N_NODES = 50000
FINAL = 41
N_PAD = 51200
ROWS_PER_SUB = 3200
CHUNK = 128
N_CHUNKS = 196
N_WORKERS = 32
E_PAD = N_WORKERS * N_CHUNKS * CHUNK
DUMP_ROW = 50000
ZC = 128
OC = 800


def _make_seg_sum(C):
    mesh = plsc.VectorSubcoreMesh(core_axis_name="c", subcore_axis_name="s")

    @functools.partial(
        pl.kernel,
        out_type=jax.ShapeDtypeStruct((2, N_PAD, C), jnp.float32),
        mesh=mesh,
        scratch_types=[
            pltpu.VMEM((N_CHUNKS, CHUNK), jnp.int32),
            pltpu.VMEM((N_CHUNKS, CHUNK), jnp.int32),
            pltpu.VMEM((CHUNK, C), jnp.float32),
            pltpu.VMEM((ZC, C), jnp.float32),
            pltpu.VMEM((OC, C), jnp.float32),
            pltpu.VMEM_SHARED((N_PAD, C), jnp.float32),
            pltpu.SemaphoreType.DMA,
        ],
    )
    def seg_sum(src_hbm, dst_hbm, h_hbm, zeros_hbm, out_hbm,
                src_v, dst_v, rows_v, zbuf, obuf, agg_sh, sem):
        cid = lax.axis_index("c")
        sid = lax.axis_index("s")
        wid = cid * 16 + sid
        row0 = sid * ROWS_PER_SUB

        pltpu.sync_copy(zeros_hbm, zbuf)

        def zbody(t, carry):
            pltpu.sync_copy(zbuf, agg_sh.at[pl.ds(row0 + t * ZC, ZC)])
            return carry

        lax.fori_loop(0, ROWS_PER_SUB // ZC, zbody, 0)
        plsc.subcore_barrier()

        pltpu.sync_copy(src_hbm.at[wid], src_v)
        pltpu.sync_copy(dst_hbm.at[wid], dst_v)

        def ebody(j, carry):
            pltpu.async_copy(h_hbm.at[src_v.at[j]], rows_v, sem).wait()
            pltpu.sync_copy(rows_v, agg_sh.at[dst_v.at[j]], add=True)
            return carry

        lax.fori_loop(0, N_CHUNKS, ebody, 0)
        plsc.subcore_barrier()

        def obody(t, carry):
            sl = pl.ds(row0 + t * OC, OC)
            pltpu.sync_copy(agg_sh.at[sl], obuf)
            pltpu.sync_copy(obuf, out_hbm.at[cid, sl])
            return carry

        lax.fori_loop(0, ROWS_PER_SUB // OC, obody, 0)

    return seg_sum


def _gelu(y):
    return 0.5 * y * (1.0 + lax.erf(y * np.float32(1.0 / np.sqrt(2.0))))


def _tc_layer(p, h, wrelT, brel2, wrootT):
    Cin, Cout = wrelT.shape
    R = 400
    G = N_NODES // R

    def body(pa, pb, h_ref, wr, br, wo, o_ref):
        agg = pa[0] + pb[0]
        if Cin == 1:
            y = agg * wr[...] + br[...] + h_ref[...] * wo[...]
        else:
            y = (jnp.dot(agg, wr[...], preferred_element_type=jnp.float32)
                 + br[...]
                 + jnp.dot(h_ref[...], wo[...],
                           preferred_element_type=jnp.float32))
        o_ref[...] = _gelu(y)

    return pl.pallas_call(
        body,
        grid=(G,),
        in_specs=[
            pl.BlockSpec((1, R, Cin), lambda i: (0, i, 0)),
            pl.BlockSpec((1, R, Cin), lambda i: (1, i, 0)),
            pl.BlockSpec((R, Cin), lambda i: (i, 0)),
            pl.BlockSpec((Cin, Cout), lambda i: (0, 0)),
            pl.BlockSpec((1, Cout), lambda i: (0, 0)),
            pl.BlockSpec((Cin, Cout), lambda i: (0, 0)),
        ],
        out_specs=pl.BlockSpec((R, Cout), lambda i: (i, 0)),
        out_shape=jax.ShapeDtypeStruct((N_NODES, Cout), jnp.float32),
    )(p, p, h, wrelT, brel2, wrootT)


def _tc_layer4_head(p, h, wrelT, brel2, wrootT, wfold):
    Cin, Cout = wrelT.shape
    R = 400
    G = N_NODES // R

    def body(pa, pb, h_ref, wr, br, wo, wf, o_ref):
        agg = pa[0] + pb[0]
        y = (jnp.dot(agg, wr[...], preferred_element_type=jnp.float32)
             + br[...]
             + jnp.dot(h_ref[...], wo[...], preferred_element_type=jnp.float32))
        h4 = _gelu(y)

        @pl.when(pl.program_id(0) == 0)
        def _init():
            o_ref[...] = jnp.zeros_like(o_ref)

        o_ref[...] += jnp.sum(h4 * wf[...]).reshape(1, 1)

    return pl.pallas_call(
        body,
        grid=(G,),
        in_specs=[
            pl.BlockSpec((1, R, Cin), lambda i: (0, i, 0)),
            pl.BlockSpec((1, R, Cin), lambda i: (1, i, 0)),
            pl.BlockSpec((R, Cin), lambda i: (i, 0)),
            pl.BlockSpec((Cin, Cout), lambda i: (0, 0)),
            pl.BlockSpec((1, Cout), lambda i: (0, 0)),
            pl.BlockSpec((Cin, Cout), lambda i: (0, 0)),
            pl.BlockSpec((R, Cout), lambda i: (i, 0)),
        ],
        out_specs=pl.BlockSpec((1, 1), lambda i: (0, 0)),
        out_shape=jax.ShapeDtypeStruct((1, 1), jnp.float32),
    )(p, p, h, wrelT, brel2, wrootT, wfold)


def _fold_head_weights(Wout):
    ar = np.arange(FINAL)
    starts = (ar * N_NODES) // FINAL
    ends = -((-((ar + 1) * N_NODES)) // FINAL)
    lens = (ends - starts).astype(np.float32)
    p = jnp.arange(N_NODES, dtype=jnp.int32)
    k0 = (FINAL * p) // N_NODES
    k1 = jnp.minimum(k0 + 1, FINAL - 1)
    in_k1 = (p >= jnp.asarray(starts, dtype=jnp.int32)[k1]) & (k1 > k0)
    Wg = Wout.reshape(8 * 8, FINAL) / jnp.asarray(lens)
    WgT = Wg.T
    w_nc = WgT[k0] + jnp.where(in_k1[:, None], WgT[k1], 0.0)
    return w_nc.T.reshape(-1).reshape(N_NODES, 8 * 8)


def kernel(x, edge_index, Wrel1, brel1, Wroot1, Wrel2, brel2, Wroot2,
           Wrel3, brel3, Wroot3, Wrel4, brel4, Wroot4, Wout, bout):
    src = edge_index[0]
    dst = edge_index[1]
    npad = E_PAD - src.shape[0]
    src3 = jnp.concatenate(
        [src, jnp.zeros((npad,), jnp.int32)]).reshape(N_WORKERS, N_CHUNKS, CHUNK)
    dst3 = jnp.concatenate(
        [dst, jnp.full((npad,), DUMP_ROW, jnp.int32)]).reshape(N_WORKERS, N_CHUNKS, CHUNK)

    zeros = {c: jnp.zeros((ZC, c), jnp.float32) for c in (1, 8, 16, 32)}
    wfold = _fold_head_weights(Wout)

    p1 = _make_seg_sum(1)(src3, dst3, x, zeros[1])
    h1 = _tc_layer(p1, x, Wrel1.T, brel1.reshape(1, -1), Wroot1.T)
    p2 = _make_seg_sum(8)(src3, dst3, h1, zeros[8])
    h2 = _tc_layer(p2, h1, Wrel2.T, brel2.reshape(1, -1), Wroot2.T)
    p3 = _make_seg_sum(16)(src3, dst3, h2, zeros[16])
    h3 = _tc_layer(p3, h2, Wrel3.T, brel3.reshape(1, -1), Wroot3.T)
    p4 = _make_seg_sum(32)(src3, dst3, h3, zeros[32])
    acc = _tc_layer4_head(p4, h3, Wrel4.T, brel4.reshape(1, -1), Wroot4.T, wfold)
    return acc.reshape(1) + bout


# speedup vs baseline: 10.4734x; 10.4734x over previous
import functools

import jax
import jax.numpy as jnp
import numpy as np
from jax import lax
from jax.experimental import pallas as pl
from jax.experimental.pallas import tpu as pltpu
from jax.experimental.pallas import tpu_sc as plsc

N_NODES = 50000
FINAL = 41
N_PAD = 51200
ROWS_PER_SUB = 3200
CHUNK = 128
N_CHUNKS = 196
N_WORKERS = 32
E_PAD = N_WORKERS * N_CHUNKS * CHUNK
DUMP_ROW = 50000
ZC = 128
OC = 200
IB = 28


def _make_seg_sum(C):
    mesh = plsc.VectorSubcoreMesh(core_axis_name="c", subcore_axis_name="s")

    @functools.partial(
        pl.kernel,
        out_type=jax.ShapeDtypeStruct((2, N_PAD, C), jnp.float32),
        mesh=mesh,
        scratch_types=[
            pltpu.VMEM((IB, CHUNK), jnp.int32),
            pltpu.VMEM((IB, CHUNK), jnp.int32),
            pltpu.VMEM((CHUNK, C), jnp.float32),
            pltpu.VMEM((OC, C), jnp.float32),
            pltpu.VMEM_SHARED((N_PAD, C), jnp.float32),
            pltpu.SemaphoreType.DMA,
        ],
        compiler_params=pltpu.CompilerParams(use_tc_tiling_on_sc=False),
    )
    def seg_sum(src_hbm, dst_hbm, h_hbm, zeros_hbm, out_hbm,
                sblk, dblk, rows_v, obuf, agg_sh, sem):
        cid = lax.axis_index("c")
        sid = lax.axis_index("s")
        wid = cid * 16 + sid
        row0 = sid * ROWS_PER_SUB

        pltpu.sync_copy(zeros_hbm, rows_v)

        def zbody(t, carry):
            pltpu.sync_copy(rows_v, agg_sh.at[pl.ds(row0 + t * ZC, ZC)])
            return carry

        lax.fori_loop(0, ROWS_PER_SUB // ZC, zbody, 0)
        plsc.subcore_barrier()

        def eblock(t, carry):
            pltpu.sync_copy(src_hbm.at[wid, pl.ds(t * IB, IB)], sblk)
            pltpu.sync_copy(dst_hbm.at[wid, pl.ds(t * IB, IB)], dblk)

            def ebody(j, c2):
                pltpu.async_copy(h_hbm.at[sblk.at[j]], rows_v, sem).wait()
                pltpu.sync_copy(rows_v, agg_sh.at[dblk.at[j]], add=True)
                return c2

            lax.fori_loop(0, IB, ebody, 0)
            return carry

        lax.fori_loop(0, N_CHUNKS // IB, eblock, 0)
        plsc.subcore_barrier()

        def obody(t, carry):
            sl = pl.ds(row0 + t * OC, OC)
            pltpu.sync_copy(agg_sh.at[sl], obuf)
            pltpu.sync_copy(obuf, out_hbm.at[cid, sl])
            return carry

        lax.fori_loop(0, ROWS_PER_SUB // OC, obody, 0)

    return seg_sum


def _gelu(y):
    return 0.5 * y * (1.0 + lax.erf(y * np.float32(1.0 / np.sqrt(2.0))))


def _tc_premul(x, wrelT):
    Cout = wrelT.shape[1]
    R = 400
    G = N_NODES // R

    def body(x_ref, w_ref, o_ref):
        o_ref[...] = x_ref[...] * w_ref[...]

    return pl.pallas_call(
        body,
        grid=(G,),
        in_specs=[
            pl.BlockSpec((R, 1), lambda i: (i, 0)),
            pl.BlockSpec((1, Cout), lambda i: (0, 0)),
        ],
        out_specs=pl.BlockSpec((R, Cout), lambda i: (i, 0)),
        out_shape=jax.ShapeDtypeStruct((N_NODES, Cout), jnp.float32),
    )(x, wrelT)


def _tc_layer1(p, x, brel2, wrootT):
    Cout = wrootT.shape[1]
    R = 400
    G = N_NODES // R

    def body(pa, pb, x_ref, br, wo, o_ref):
        y = pa[0] + pb[0] + br[...] + x_ref[...] * wo[...]
        o_ref[...] = _gelu(y)

    return pl.pallas_call(
        body,
        grid=(G,),
        in_specs=[
            pl.BlockSpec((1, R, Cout), lambda i: (0, i, 0)),
            pl.BlockSpec((1, R, Cout), lambda i: (1, i, 0)),
            pl.BlockSpec((R, 1), lambda i: (i, 0)),
            pl.BlockSpec((1, Cout), lambda i: (0, 0)),
            pl.BlockSpec((1, Cout), lambda i: (0, 0)),
        ],
        out_specs=pl.BlockSpec((R, Cout), lambda i: (i, 0)),
        out_shape=jax.ShapeDtypeStruct((N_NODES, Cout), jnp.float32),
    )(p, p, x, brel2, wrootT)


def _tc_layer(p, h, wrelT, brel2, wrootT):
    Cin, Cout = wrelT.shape
    R = 400
    G = N_NODES // R

    def body(pa, pb, h_ref, wr, br, wo, o_ref):
        agg = pa[0] + pb[0]
        y = (jnp.dot(agg, wr[...], preferred_element_type=jnp.float32)
             + br[...]
             + jnp.dot(h_ref[...], wo[...],
                       preferred_element_type=jnp.float32))
        o_ref[...] = _gelu(y)

    return pl.pallas_call(
        body,
        grid=(G,),
        in_specs=[
            pl.BlockSpec((1, R, Cin), lambda i: (0, i, 0)),
            pl.BlockSpec((1, R, Cin), lambda i: (1, i, 0)),
            pl.BlockSpec((R, Cin), lambda i: (i, 0)),
            pl.BlockSpec((Cin, Cout), lambda i: (0, 0)),
            pl.BlockSpec((1, Cout), lambda i: (0, 0)),
            pl.BlockSpec((Cin, Cout), lambda i: (0, 0)),
        ],
        out_specs=pl.BlockSpec((R, Cout), lambda i: (i, 0)),
        out_shape=jax.ShapeDtypeStruct((N_NODES, Cout), jnp.float32),
    )(p, p, h, wrelT, brel2, wrootT)


def _tc_layer4_head(p, h, wrelT, brel2, wrootT, wfold):
    Cin, Cout = wrelT.shape
    R = 400
    G = N_NODES // R

    def body(pa, pb, h_ref, wr, br, wo, wf, o_ref):
        agg = pa[0] + pb[0]
        y = (jnp.dot(agg, wr[...], preferred_element_type=jnp.float32)
             + br[...]
             + jnp.dot(h_ref[...], wo[...], preferred_element_type=jnp.float32))
        h4 = _gelu(y)

        @pl.when(pl.program_id(0) == 0)
        def _init():
            o_ref[...] = jnp.zeros_like(o_ref)

        o_ref[...] += jnp.sum(h4 * wf[...]).reshape(1, 1)

    return pl.pallas_call(
        body,
        grid=(G,),
        in_specs=[
            pl.BlockSpec((1, R, Cin), lambda i: (0, i, 0)),
            pl.BlockSpec((1, R, Cin), lambda i: (1, i, 0)),
            pl.BlockSpec((R, Cin), lambda i: (i, 0)),
            pl.BlockSpec((Cin, Cout), lambda i: (0, 0)),
            pl.BlockSpec((1, Cout), lambda i: (0, 0)),
            pl.BlockSpec((Cin, Cout), lambda i: (0, 0)),
            pl.BlockSpec((R, Cout), lambda i: (i, 0)),
        ],
        out_specs=pl.BlockSpec((1, 1), lambda i: (0, 0)),
        out_shape=jax.ShapeDtypeStruct((1, 1), jnp.float32),
    )(p, p, h, wrelT, brel2, wrootT, wfold)


def _fold_head_weights(Wout):
    ar = np.arange(FINAL)
    starts = (ar * N_NODES) // FINAL
    ends = -((-((ar + 1) * N_NODES)) // FINAL)
    lens = (ends - starts).astype(np.float32)
    p = jnp.arange(N_NODES, dtype=jnp.int32)
    k0 = (FINAL * p) // N_NODES
    k1 = jnp.minimum(k0 + 1, FINAL - 1)
    in_k1 = (p >= jnp.asarray(starts, dtype=jnp.int32)[k1]) & (k1 > k0)
    Wg = Wout.reshape(8 * 8, FINAL) / jnp.asarray(lens)
    WgT = Wg.T
    w_nc = WgT[k0] + jnp.where(in_k1[:, None], WgT[k1], 0.0)
    return w_nc.T.reshape(-1).reshape(N_NODES, 8 * 8)


def kernel(x, edge_index, Wrel1, brel1, Wroot1, Wrel2, brel2, Wroot2,
           Wrel3, brel3, Wroot3, Wrel4, brel4, Wroot4, Wout, bout):
    src = edge_index[0]
    dst = edge_index[1]
    npad = E_PAD - src.shape[0]
    src3 = jnp.concatenate(
        [src, jnp.zeros((npad,), jnp.int32)]).reshape(N_WORKERS, N_CHUNKS, CHUNK)
    dst3 = jnp.concatenate(
        [dst, jnp.full((npad,), DUMP_ROW, jnp.int32)]).reshape(N_WORKERS, N_CHUNKS, CHUNK)

    zeros = {c: jnp.zeros((ZC, c), jnp.float32) for c in (8, 16, 32)}
    wfold = _fold_head_weights(Wout)

    g1 = _tc_premul(x, Wrel1.T)
    p1 = _make_seg_sum(8)(src3, dst3, g1, zeros[8])
    h1 = _tc_layer1(p1, x, brel1.reshape(1, -1), Wroot1.T)
    p2 = _make_seg_sum(8)(src3, dst3, h1, zeros[8])
    h2 = _tc_layer(p2, h1, Wrel2.T, brel2.reshape(1, -1), Wroot2.T)
    p3 = _make_seg_sum(16)(src3, dst3, h2, zeros[16])
    h3 = _tc_layer(p3, h2, Wrel3.T, brel3.reshape(1, -1), Wroot3.T)
    p4 = _make_seg_sum(32)(src3, dst3, h3, zeros[32])
    acc = _tc_layer4_head(p4, h3, Wrel4.T, brel4.reshape(1, -1), Wroot4.T, wfold)
    return acc.reshape(1) + bout


# speedup vs baseline: 11.7193x; 1.1190x over previous
import functools

import jax
import jax.numpy as jnp
import numpy as np
from jax import lax
from jax.experimental import pallas as pl
from jax.experimental.pallas import tpu as pltpu
from jax.experimental.pallas import tpu_sc as plsc

N_NODES = 50000
FINAL = 41
N_PAD = 50048
ROWS_PER_SUB = 3128
N_WORKERS = 32
E_PER_W = 25344
E_PAD = N_WORKERS * E_PER_W
DUMP_ROW = 50000
_CFG = {8: (528, 48, 12), 16: (528, 48, 12), 32: (132, 192, 48)}
NBUF = 4


def _make_seg_sum(C):
    CHUNK, NCH, IBC = _CFG[C]
    mesh = plsc.VectorSubcoreMesh(core_axis_name="c", subcore_axis_name="s")

    @functools.partial(
        pl.kernel,
        out_type=jax.ShapeDtypeStruct((2, N_PAD, C), jnp.float32),
        mesh=mesh,
        scratch_types=[
            pltpu.VMEM((IBC, CHUNK), jnp.int32),
            pltpu.VMEM((IBC, CHUNK), jnp.int32),
            pltpu.VMEM((CHUNK, C), jnp.float32),
            pltpu.VMEM((CHUNK, C), jnp.float32),
            pltpu.VMEM((CHUNK, C), jnp.float32),
            pltpu.VMEM((CHUNK, C), jnp.float32),
            pltpu.VMEM_SHARED((N_PAD, C), jnp.float32),
            pltpu.SemaphoreType.DMA((NBUF,)),
            pltpu.SemaphoreType.DMA((NBUF,)),
        ],
        compiler_params=pltpu.CompilerParams(use_tc_tiling_on_sc=False),
    )
    def seg_sum(src_hbm, dst_hbm, h_hbm, zeros_hbm, out_hbm,
                sblk, dblk, r0, r1, r2, r3, agg_sh, gsem, ssem):
        rows = [r0, r1, r2, r3]
        cid = lax.axis_index("c")
        sid = lax.axis_index("s")
        wid = cid * 16 + sid
        sl_me = pl.ds(sid * ROWS_PER_SUB, ROWS_PER_SUB)

        def g_start(j, b):
            pltpu.async_copy(h_hbm.at[sblk.at[j]], rows[b], gsem.at[b])

        def g_wait(j, b):
            pltpu.make_async_copy(h_hbm.at[sblk.at[j]], rows[b],
                                  gsem.at[b]).wait()

        def s_start(j, b):
            pltpu.async_copy(rows[b], agg_sh.at[dblk.at[j]], ssem.at[b],
                             add=True)

        def s_wait(j, b):
            pltpu.make_async_copy(rows[b], agg_sh.at[dblk.at[j]],
                                  ssem.at[b]).wait()

        pltpu.sync_copy(zeros_hbm, agg_sh.at[sl_me])
        plsc.subcore_barrier()

        for t in range(NCH // IBC):
            pltpu.sync_copy(src_hbm.at[wid, pl.ds(t * IBC, IBC)], sblk)
            pltpu.sync_copy(dst_hbm.at[wid, pl.ds(t * IBC, IBC)], dblk)
            for b in range(NBUF):
                g_start(b, b)
            for j in range(2):
                g_wait(j, j)
                s_start(j, j)

            def grp(i, carry):
                j0 = 2 + i * NBUF
                for q in range(NBUF):
                    j = j0 + q
                    b = (2 + q) % NBUF
                    g_wait(j, b)
                    s_start(j, b)
                    b2 = (q + 4) % NBUF
                    s_wait(j - 2, b2)
                    g_start(j + 2, b2)
                return carry

            lax.fori_loop(0, (IBC - 4) // NBUF, grp, 0)

            for j in (IBC - 2, IBC - 1):
                b = j % NBUF
                g_wait(j, b)
                s_start(j, b)
            for b in range(NBUF):
                s_wait(0, b)
        plsc.subcore_barrier()

        pltpu.sync_copy(agg_sh.at[sl_me], out_hbm.at[cid, sl_me])

    return seg_sum


def _gelu(y):
    return 0.5 * y * (1.0 + lax.erf(y * np.float32(1.0 / np.sqrt(2.0))))


def _tc_premul(x, wrelT):
    Cout = wrelT.shape[1]
    R = 400
    G = N_NODES // R

    def body(x_ref, w_ref, o_ref):
        o_ref[...] = x_ref[...] * w_ref[...]

    return pl.pallas_call(
        body,
        grid=(G,),
        in_specs=[
            pl.BlockSpec((R, 1), lambda i: (i, 0)),
            pl.BlockSpec((1, Cout), lambda i: (0, 0)),
        ],
        out_specs=pl.BlockSpec((R, Cout), lambda i: (i, 0)),
        out_shape=jax.ShapeDtypeStruct((N_NODES, Cout), jnp.float32),
    )(x, wrelT)


def _tc_layer1(p, x, brel2, wrootT):
    Cout = wrootT.shape[1]
    R = 400
    G = N_NODES // R

    def body(pa, pb, x_ref, br, wo, o_ref):
        y = pa[0] + pb[0] + br[...] + x_ref[...] * wo[...]
        o_ref[...] = _gelu(y)

    return pl.pallas_call(
        body,
        grid=(G,),
        in_specs=[
            pl.BlockSpec((1, R, Cout), lambda i: (0, i, 0)),
            pl.BlockSpec((1, R, Cout), lambda i: (1, i, 0)),
            pl.BlockSpec((R, 1), lambda i: (i, 0)),
            pl.BlockSpec((1, Cout), lambda i: (0, 0)),
            pl.BlockSpec((1, Cout), lambda i: (0, 0)),
        ],
        out_specs=pl.BlockSpec((R, Cout), lambda i: (i, 0)),
        out_shape=jax.ShapeDtypeStruct((N_NODES, Cout), jnp.float32),
    )(p, p, x, brel2, wrootT)


def _tc_layer(p, h, wrelT, brel2, wrootT):
    Cin, Cout = wrelT.shape
    R = 400
    G = N_NODES // R

    def body(pa, pb, h_ref, wr, br, wo, o_ref):
        agg = pa[0] + pb[0]
        y = (jnp.dot(agg, wr[...], preferred_element_type=jnp.float32)
             + br[...]
             + jnp.dot(h_ref[...], wo[...],
                       preferred_element_type=jnp.float32))
        o_ref[...] = _gelu(y)

    return pl.pallas_call(
        body,
        grid=(G,),
        in_specs=[
            pl.BlockSpec((1, R, Cin), lambda i: (0, i, 0)),
            pl.BlockSpec((1, R, Cin), lambda i: (1, i, 0)),
            pl.BlockSpec((R, Cin), lambda i: (i, 0)),
            pl.BlockSpec((Cin, Cout), lambda i: (0, 0)),
            pl.BlockSpec((1, Cout), lambda i: (0, 0)),
            pl.BlockSpec((Cin, Cout), lambda i: (0, 0)),
        ],
        out_specs=pl.BlockSpec((R, Cout), lambda i: (i, 0)),
        out_shape=jax.ShapeDtypeStruct((N_NODES, Cout), jnp.float32),
    )(p, p, h, wrelT, brel2, wrootT)


def _tc_layer4_head(p, h, wrelT, brel2, wrootT, wfold):
    Cin, Cout = wrelT.shape
    R = 400
    G = N_NODES // R

    def body(pa, pb, h_ref, wr, br, wo, wf, o_ref):
        agg = pa[0] + pb[0]
        y = (jnp.dot(agg, wr[...], preferred_element_type=jnp.float32)
             + br[...]
             + jnp.dot(h_ref[...], wo[...], preferred_element_type=jnp.float32))
        h4 = _gelu(y)

        @pl.when(pl.program_id(0) == 0)
        def _init():
            o_ref[...] = jnp.zeros_like(o_ref)

        o_ref[...] += jnp.sum(h4 * wf[...]).reshape(1, 1)

    return pl.pallas_call(
        body,
        grid=(G,),
        in_specs=[
            pl.BlockSpec((1, R, Cin), lambda i: (0, i, 0)),
            pl.BlockSpec((1, R, Cin), lambda i: (1, i, 0)),
            pl.BlockSpec((R, Cin), lambda i: (i, 0)),
            pl.BlockSpec((Cin, Cout), lambda i: (0, 0)),
            pl.BlockSpec((1, Cout), lambda i: (0, 0)),
            pl.BlockSpec((Cin, Cout), lambda i: (0, 0)),
            pl.BlockSpec((R, Cout), lambda i: (i, 0)),
        ],
        out_specs=pl.BlockSpec((1, 1), lambda i: (0, 0)),
        out_shape=jax.ShapeDtypeStruct((1, 1), jnp.float32),
    )(p, p, h, wrelT, brel2, wrootT, wfold)


def _fold_head_weights(Wout):
    ar = np.arange(FINAL)
    starts = (ar * N_NODES) // FINAL
    ends = -((-((ar + 1) * N_NODES)) // FINAL)
    lens = (ends - starts).astype(np.float32)
    p = jnp.arange(N_NODES, dtype=jnp.int32)
    k0 = (FINAL * p) // N_NODES
    k1 = jnp.minimum(k0 + 1, FINAL - 1)
    in_k1 = (p >= jnp.asarray(starts, dtype=jnp.int32)[k1]) & (k1 > k0)
    Wg = Wout.reshape(8 * 8, FINAL) / jnp.asarray(lens)
    WgT = Wg.T
    w_nc = WgT[k0] + jnp.where(in_k1[:, None], WgT[k1], 0.0)
    return w_nc.T.reshape(-1).reshape(N_NODES, 8 * 8)


def kernel(x, edge_index, Wrel1, brel1, Wroot1, Wrel2, brel2, Wroot2,
           Wrel3, brel3, Wroot3, Wrel4, brel4, Wroot4, Wout, bout):
    src = edge_index[0]
    dst = edge_index[1]
    npad = E_PAD - src.shape[0]
    src_p = jnp.concatenate([src, jnp.zeros((npad,), jnp.int32)])
    dst_p = jnp.concatenate([dst, jnp.full((npad,), DUMP_ROW, jnp.int32)])
    srcA = src_p.reshape(N_WORKERS, _CFG[8][1], _CFG[8][0])
    dstA = dst_p.reshape(N_WORKERS, _CFG[8][1], _CFG[8][0])
    srcB = src_p.reshape(N_WORKERS, _CFG[32][1], _CFG[32][0])
    dstB = dst_p.reshape(N_WORKERS, _CFG[32][1], _CFG[32][0])

    zeros = {c: jnp.zeros((ROWS_PER_SUB, c), jnp.float32) for c in (8, 16, 32)}
    wfold = _fold_head_weights(Wout)

    g1 = _tc_premul(x, Wrel1.T)
    p1 = _make_seg_sum(8)(srcA, dstA, g1, zeros[8])
    h1 = _tc_layer1(p1, x, brel1.reshape(1, -1), Wroot1.T)
    p2 = _make_seg_sum(8)(srcA, dstA, h1, zeros[8])
    h2 = _tc_layer(p2, h1, Wrel2.T, brel2.reshape(1, -1), Wroot2.T)
    p3 = _make_seg_sum(16)(srcA, dstA, h2, zeros[16])
    h3 = _tc_layer(p3, h2, Wrel3.T, brel3.reshape(1, -1), Wroot3.T)
    p4 = _make_seg_sum(32)(srcB, dstB, h3, zeros[32])
    acc = _tc_layer4_head(p4, h3, Wrel4.T, brel4.reshape(1, -1), Wroot4.T, wfold)
    return acc.reshape(1) + bout


# speedup vs baseline: 17.3150x; 1.4775x over previous
import functools

import jax
import jax.numpy as jnp
import numpy as np
from jax import lax
from jax.experimental import pallas as pl
from jax.experimental.pallas import tpu as pltpu
from jax.experimental.pallas import tpu_sc as plsc

N_NODES = 50000
FINAL = 41
N_PAD = 50048
NP = N_PAD // 16
ROWS_PER_SUB = 3128
N_WORKERS = 32
E_PER_W = 25600
E_PAD = N_WORKERS * E_PER_W
DUMP_ROW = 50000
_CFG = {8: (512, 50, 25), 16: (512, 50, 25), 32: (128, 200, 40)}
NBUF = 4


def _make_seg_sum(C):
    CHUNK, NCH, IBC = _CFG[C]
    mesh = plsc.VectorSubcoreMesh(core_axis_name="c", subcore_axis_name="s")

    @functools.partial(
        pl.kernel,
        out_type=jax.ShapeDtypeStruct((2, N_PAD, C), jnp.float32),
        mesh=mesh,
        scratch_types=[
            pltpu.VMEM((IBC, CHUNK), jnp.int32),
            pltpu.VMEM((IBC, CHUNK), jnp.int32),
            pltpu.VMEM((CHUNK, C), jnp.float32),
            pltpu.VMEM((CHUNK, C), jnp.float32),
            pltpu.VMEM((CHUNK, C), jnp.float32),
            pltpu.VMEM((CHUNK, C), jnp.float32),
            pltpu.VMEM_SHARED((N_PAD, C), jnp.float32),
            pltpu.SemaphoreType.DMA((NBUF,)),
            pltpu.SemaphoreType.DMA((NBUF,)),
        ],
        compiler_params=pltpu.CompilerParams(use_tc_tiling_on_sc=False),
    )
    def seg_sum(src_hbm, dst_hbm, h_hbm, zeros_hbm, out_hbm,
                sblk, dblk, r0, r1, r2, r3, agg_sh, gsem, ssem):
        rows = [r0, r1, r2, r3]
        cid = lax.axis_index("c")
        sid = lax.axis_index("s")
        wid = cid * 16 + sid
        sl_me = pl.ds(sid * ROWS_PER_SUB, ROWS_PER_SUB)

        def g_start(j, b):
            pltpu.async_copy(h_hbm.at[sblk.at[j]], rows[b], gsem.at[b])

        def g_wait(j, b):
            pltpu.make_async_copy(h_hbm.at[sblk.at[j]], rows[b],
                                  gsem.at[b]).wait()

        def s_start(j, b):
            pltpu.async_copy(rows[b], agg_sh.at[dblk.at[j]], ssem.at[b],
                             add=True)

        def s_wait(j, b):
            pltpu.make_async_copy(rows[b], agg_sh.at[dblk.at[j]],
                                  ssem.at[b]).wait()

        pltpu.sync_copy(zeros_hbm, agg_sh.at[sl_me])
        plsc.subcore_barrier()

        for t in range(NCH // IBC):
            blk0 = wid * NCH + t * IBC
            pltpu.sync_copy(src_hbm.at[pl.ds(blk0, IBC)], sblk)
            pltpu.sync_copy(dst_hbm.at[pl.ds(blk0, IBC)], dblk)
            for b in range(NBUF):
                g_start(b, b)
            for j in range(2):
                g_wait(j, j)
                s_start(j, j)

            n_grp = (IBC - 6) // NBUF

            def grp(i, carry):
                j0 = 2 + i * NBUF
                for q in range(NBUF):
                    j = j0 + q
                    b = (2 + q) % NBUF
                    g_wait(j, b)
                    s_start(j, b)
                    b2 = q
                    s_wait(j - 2, b2)
                    g_start(j + 2, b2)
                return carry

            lax.fori_loop(0, n_grp, grp, 0)

            for j in range(2 + n_grp * NBUF, IBC):
                b = j % NBUF
                g_wait(j, b)
                s_start(j, b)
                if j + 2 < IBC:
                    s_wait(j - 2, (j + 2) % NBUF)
                    g_start(j + 2, (j + 2) % NBUF)
            for b in range(NBUF):
                s_wait(0, b)
        plsc.subcore_barrier()

        pltpu.sync_copy(agg_sh.at[sl_me], out_hbm.at[cid, sl_me])

    return seg_sum


def _gelu(y):
    return 0.5 * y * (1.0 + lax.erf(y * np.float32(1.0 / np.sqrt(2.0))))


def _kron16(w):
    return jnp.kron(jnp.eye(16, dtype=jnp.float32), w)


BLK = 184
GRID = NP // BLK


def _tc_premul(x2, wx):
    def body(x_ref, w_ref, o_ref):
        o_ref[...] = jnp.dot(x_ref[...], w_ref[...],
                             preferred_element_type=jnp.float32, precision=lax.Precision.HIGHEST)

    return pl.pallas_call(
        body,
        grid=(GRID,),
        in_specs=[
            pl.BlockSpec((BLK, 16), lambda i: (i, 0)),
            pl.BlockSpec((16, 128), lambda i: (0, 0)),
        ],
        out_specs=pl.BlockSpec((BLK, 128), lambda i: (i, 0)),
        out_shape=jax.ShapeDtypeStruct((NP, 128), jnp.float32),
    )(x2, wx)


def _tc_layer1(p, x2, brelT, wrootx):
    def body(p_ref, x_ref, br, wo, o_ref):
        y = (p_ref[0] + p_ref[1] + br[...]
             + jnp.dot(x_ref[...], wo[...],
                       preferred_element_type=jnp.float32, precision=lax.Precision.HIGHEST))
        o_ref[...] = _gelu(y)

    return pl.pallas_call(
        body,
        grid=(GRID,),
        in_specs=[
            pl.BlockSpec((2, BLK, 128), lambda i: (0, i, 0)),
            pl.BlockSpec((BLK, 16), lambda i: (i, 0)),
            pl.BlockSpec((1, 128), lambda i: (0, 0)),
            pl.BlockSpec((16, 128), lambda i: (0, 0)),
        ],
        out_specs=pl.BlockSpec((BLK, 128), lambda i: (i, 0)),
        out_shape=jax.ShapeDtypeStruct((NP, 128), jnp.float32),
    )(p, x2, brelT, wrootx)


def _tc_layer(p, h, wrelK, brelT, wrootK):
    Cin16, Cout16 = wrelK.shape

    def body(p_ref, h_ref, wr, br, wo, o_ref):
        agg = p_ref[0] + p_ref[1]
        y = (jnp.dot(agg, wr[...], preferred_element_type=jnp.float32, precision=lax.Precision.HIGHEST)
             + br[...]
             + jnp.dot(h_ref[...], wo[...],
                       preferred_element_type=jnp.float32, precision=lax.Precision.HIGHEST))
        o_ref[...] = _gelu(y)

    return pl.pallas_call(
        body,
        grid=(GRID,),
        in_specs=[
            pl.BlockSpec((2, BLK, Cin16), lambda i: (0, i, 0)),
            pl.BlockSpec((BLK, Cin16), lambda i: (i, 0)),
            pl.BlockSpec((Cin16, Cout16), lambda i: (0, 0)),
            pl.BlockSpec((1, Cout16), lambda i: (0, 0)),
            pl.BlockSpec((Cin16, Cout16), lambda i: (0, 0)),
        ],
        out_specs=pl.BlockSpec((BLK, Cout16), lambda i: (i, 0)),
        out_shape=jax.ShapeDtypeStruct((NP, Cout16), jnp.float32),
    )(p, h, wrelK, brelT, wrootK)


def _tc_head(p, h, wrelK, brelT, wrootK, wpack):
    def body(p_ref, h_ref, wr, br, wo, wp, o_ref):
        agg = p_ref[0] + p_ref[1]
        y = (jnp.dot(agg, wr[...], preferred_element_type=jnp.float32, precision=lax.Precision.HIGHEST)
             + br[...]
             + jnp.dot(h_ref[...], wo[...],
                       preferred_element_type=jnp.float32, precision=lax.Precision.HIGHEST))
        h4 = _gelu(y)

        @pl.when(pl.program_id(0) == 0)
        def _init():
            o_ref[...] = jnp.zeros_like(o_ref)

        o_ref[...] += jnp.sum(h4 * wp[...]).reshape(1, 1)

    return pl.pallas_call(
        body,
        grid=(GRID,),
        in_specs=[
            pl.BlockSpec((2, BLK, 512), lambda i: (0, i, 0)),
            pl.BlockSpec((BLK, 512), lambda i: (i, 0)),
            pl.BlockSpec((512, 1024), lambda i: (0, 0)),
            pl.BlockSpec((1, 1024), lambda i: (0, 0)),
            pl.BlockSpec((512, 1024), lambda i: (0, 0)),
            pl.BlockSpec((BLK, 1024), lambda i: (i, 0)),
        ],
        out_specs=pl.BlockSpec((1, 1), lambda i: (0, 0)),
        out_shape=jax.ShapeDtypeStruct((1, 1), jnp.float32),
    )(p, h, wrelK, brelT, wrootK, wpack)


def _bin_membership():
    ar = np.arange(FINAL)
    starts = (ar * N_NODES) // FINAL
    ends = -((-((ar + 1) * N_NODES)) // FINAL)
    p = np.arange(N_NODES)
    m = (p[None, :] >= starts[:, None]) & (p[None, :] < ends[:, None])
    return m.astype(np.float32) / (ends - starts)[:, None].astype(np.float32)


_MPOOL = _bin_membership()


def _head_weights(Wout):
    wv = jnp.dot(Wout.reshape(8 * 8, FINAL), jnp.asarray(_MPOOL),
                 preferred_element_type=jnp.float32, precision=lax.Precision.HIGHEST)
    flat = jnp.concatenate(
        [wv.reshape(-1), jnp.zeros(((N_PAD - N_NODES) * 64,), jnp.float32)])
    return flat.reshape(NP, 16 * 64)


def kernel(x, edge_index, Wrel1, brel1, Wroot1, Wrel2, brel2, Wroot2,
           Wrel3, brel3, Wroot3, Wrel4, brel4, Wroot4, Wout, bout):
    src = edge_index[0]
    dst = edge_index[1]
    npad = E_PAD - src.shape[0]
    src_p = jnp.concatenate([src, jnp.zeros((npad,), jnp.int32)])
    dst_p = jnp.concatenate([dst, jnp.full((npad,), DUMP_ROW, jnp.int32)])
    cfgA, cfgB = _CFG[8], _CFG[32]
    srcA = src_p.reshape(N_WORKERS * cfgA[1], cfgA[0])
    dstA = dst_p.reshape(N_WORKERS * cfgA[1], cfgA[0])
    srcB = src_p.reshape(N_WORKERS * cfgB[1], cfgB[0])
    dstB = dst_p.reshape(N_WORKERS * cfgB[1], cfgB[0])

    zeros = {c: jnp.zeros((ROWS_PER_SUB, c), jnp.float32) for c in (8, 16, 32)}
    x2 = jnp.pad(x[:, 0], (0, N_PAD - N_NODES)).reshape(NP, 16)
    wpack = _head_weights(Wout)

    g1 = _tc_premul(x2, _kron16(Wrel1.T))
    p1 = _make_seg_sum(8)(srcA, dstA, g1.reshape(N_PAD, 8), zeros[8])
    h1 = _tc_layer1(p1.reshape(2, NP, 128), x2,
                    jnp.tile(brel1, 16).reshape(1, 128), _kron16(Wroot1.T))
    p2 = _make_seg_sum(8)(srcA, dstA, h1.reshape(N_PAD, 8), zeros[8])
    h2 = _tc_layer(p2.reshape(2, NP, 128), h1, _kron16(Wrel2.T),
                   jnp.tile(brel2, 16).reshape(1, 256), _kron16(Wroot2.T))
    p3 = _make_seg_sum(16)(srcA, dstA, h2.reshape(N_PAD, 16), zeros[16])
    h3 = _tc_layer(p3.reshape(2, NP, 256), h2, _kron16(Wrel3.T),
                   jnp.tile(brel3, 16).reshape(1, 512), _kron16(Wroot3.T))
    p4 = _make_seg_sum(32)(srcB, dstB, h3.reshape(N_PAD, 32), zeros[32])
    acc = _tc_head(p4.reshape(2, NP, 512), h3, _kron16(Wrel4.T),
                   jnp.tile(brel4, 16).reshape(1, 1024), _kron16(Wroot4.T),
                   wpack)
    return acc.reshape(1) + bout


# speedup vs baseline: 18.0877x; 1.0446x over previous
import functools

import jax
import jax.numpy as jnp
import numpy as np
from jax import lax
from jax.experimental import pallas as pl
from jax.experimental.pallas import tpu as pltpu
from jax.experimental.pallas import tpu_sc as plsc

N_NODES = 50000
FINAL = 41
N_PAD = 50048
NP = N_PAD // 16
ROWS_PER_SUB = 3128
N_WORKERS = 32
E_PER_W = 25600
E_PAD = N_WORKERS * E_PER_W
DUMP_ROW = 50000
_CFG = {8: (512, 70, 35, 30, 30),
        16: (512, 70, 35, 30, 30),
        32: (128, 280, 35, 120, 30)}
_FAST_CID = 0
NBUF = 4


def _make_seg_sum(C):
    CHUNK, NCHF, IBCF, NCHS, IBCS = _CFG[C]
    IBC = max(IBCF, IBCS)
    mesh = plsc.VectorSubcoreMesh(core_axis_name="c", subcore_axis_name="s")

    @functools.partial(
        pl.kernel,
        out_type=jax.ShapeDtypeStruct((2, N_PAD, C), jnp.float32),
        mesh=mesh,
        scratch_types=[
            pltpu.VMEM((IBC, CHUNK), jnp.int32),
            pltpu.VMEM((IBC, CHUNK), jnp.int32),
            pltpu.VMEM((CHUNK, C), jnp.float32),
            pltpu.VMEM((CHUNK, C), jnp.float32),
            pltpu.VMEM((CHUNK, C), jnp.float32),
            pltpu.VMEM((CHUNK, C), jnp.float32),
            pltpu.VMEM_SHARED((N_PAD, C), jnp.float32),
            pltpu.SemaphoreType.DMA((NBUF,)),
            pltpu.SemaphoreType.DMA((NBUF,)),
        ],
        compiler_params=pltpu.CompilerParams(use_tc_tiling_on_sc=False),
    )
    def seg_sum(src_hbm, dst_hbm, h_hbm, zeros_hbm, out_hbm,
                sblk, dblk, r0, r1, r2, r3, agg_sh, gsem, ssem):
        rows = [r0, r1, r2, r3]
        cid = lax.axis_index("c")
        sid = lax.axis_index("s")
        wid = cid * 16 + sid
        sl_me = pl.ds(sid * ROWS_PER_SUB, ROWS_PER_SUB)

        def g_start(j, b):
            pltpu.async_copy(h_hbm.at[sblk.at[j]], rows[b], gsem.at[b])

        def g_wait(j, b):
            pltpu.make_async_copy(h_hbm.at[sblk.at[j]], rows[b],
                                  gsem.at[b]).wait()

        def s_start(j, b):
            pltpu.async_copy(rows[b], agg_sh.at[dblk.at[j]], ssem.at[b],
                             add=True)

        def s_wait(j, b):
            pltpu.make_async_copy(rows[b], agg_sh.at[dblk.at[j]],
                                  ssem.at[b]).wait()

        pltpu.sync_copy(zeros_hbm, agg_sh.at[sl_me])
        plsc.subcore_barrier()

        def run_block(blk0, ibc):
            pltpu.sync_copy(src_hbm.at[pl.ds(blk0, ibc)], sblk.at[pl.ds(0, ibc)])
            pltpu.sync_copy(dst_hbm.at[pl.ds(blk0, ibc)], dblk.at[pl.ds(0, ibc)])
            for b in range(NBUF):
                g_start(b, b)
            for j in range(2):
                g_wait(j, j)
                s_start(j, j)

            n_grp = (ibc - 6) // NBUF

            def grp(i, carry):
                j0 = 2 + i * NBUF
                for q in range(NBUF):
                    j = j0 + q
                    b = (2 + q) % NBUF
                    g_wait(j, b)
                    s_start(j, b)
                    b2 = q
                    s_wait(j - 2, b2)
                    g_start(j + 2, b2)
                return carry

            lax.fori_loop(0, n_grp, grp, 0)

            for j in range(2 + n_grp * NBUF, ibc):
                b = j % NBUF
                g_wait(j, b)
                s_start(j, b)
                if j + 2 < ibc:
                    s_wait(j - 2, (j + 2) % NBUF)
                    g_start(j + 2, (j + 2) % NBUF)
            for b in range(NBUF):
                s_wait(0, b)

        fast_row0 = 0 if _FAST_CID == 0 else 16 * NCHS
        slow_row0 = 16 * NCHF if _FAST_CID == 0 else 0

        @pl.when(cid == _FAST_CID)
        def _fast_side():
            for t in range(NCHF // IBCF):
                run_block(fast_row0 + sid * NCHF + t * IBCF, IBCF)

        @pl.when(cid != _FAST_CID)
        def _slow_side():
            for t in range(NCHS // IBCS):
                run_block(slow_row0 + sid * NCHS + t * IBCS, IBCS)

        plsc.subcore_barrier()

        pltpu.sync_copy(agg_sh.at[sl_me], out_hbm.at[cid, sl_me])

    return seg_sum


def _gelu(y):
    return 0.5 * y * (1.0 + lax.erf(y * np.float32(1.0 / np.sqrt(2.0))))


def _kron16(w):
    return jnp.kron(jnp.eye(16, dtype=jnp.float32), w)


BLK = 184
GRID = NP // BLK


def _tc_premul(x2, wx):
    def body(x_ref, w_ref, o_ref):
        o_ref[...] = jnp.dot(x_ref[...], w_ref[...],
                             preferred_element_type=jnp.float32, precision=lax.Precision.HIGHEST)

    return pl.pallas_call(
        body,
        grid=(GRID,),
        in_specs=[
            pl.BlockSpec((BLK, 16), lambda i: (i, 0)),
            pl.BlockSpec((16, 128), lambda i: (0, 0)),
        ],
        out_specs=pl.BlockSpec((BLK, 128), lambda i: (i, 0)),
        out_shape=jax.ShapeDtypeStruct((NP, 128), jnp.float32),
    )(x2, wx)


def _tc_layer1(p, x2, brelT, wrootx):
    def body(p_ref, x_ref, br, wo, o_ref):
        y = (p_ref[0] + p_ref[1] + br[...]
             + jnp.dot(x_ref[...], wo[...],
                       preferred_element_type=jnp.float32, precision=lax.Precision.HIGHEST))
        o_ref[...] = _gelu(y)

    return pl.pallas_call(
        body,
        grid=(GRID,),
        in_specs=[
            pl.BlockSpec((2, BLK, 128), lambda i: (0, i, 0)),
            pl.BlockSpec((BLK, 16), lambda i: (i, 0)),
            pl.BlockSpec((1, 128), lambda i: (0, 0)),
            pl.BlockSpec((16, 128), lambda i: (0, 0)),
        ],
        out_specs=pl.BlockSpec((BLK, 128), lambda i: (i, 0)),
        out_shape=jax.ShapeDtypeStruct((NP, 128), jnp.float32),
    )(p, x2, brelT, wrootx)


def _tc_layer(p, h, wrelK, brelT, wrootK):
    Cin16, Cout16 = wrelK.shape

    def body(p_ref, h_ref, wr, br, wo, o_ref):
        agg = p_ref[0] + p_ref[1]
        y = (jnp.dot(agg, wr[...], preferred_element_type=jnp.float32, precision=lax.Precision.HIGHEST)
             + br[...]
             + jnp.dot(h_ref[...], wo[...],
                       preferred_element_type=jnp.float32, precision=lax.Precision.HIGHEST))
        o_ref[...] = _gelu(y)

    return pl.pallas_call(
        body,
        grid=(GRID,),
        in_specs=[
            pl.BlockSpec((2, BLK, Cin16), lambda i: (0, i, 0)),
            pl.BlockSpec((BLK, Cin16), lambda i: (i, 0)),
            pl.BlockSpec((Cin16, Cout16), lambda i: (0, 0)),
            pl.BlockSpec((1, Cout16), lambda i: (0, 0)),
            pl.BlockSpec((Cin16, Cout16), lambda i: (0, 0)),
        ],
        out_specs=pl.BlockSpec((BLK, Cout16), lambda i: (i, 0)),
        out_shape=jax.ShapeDtypeStruct((NP, Cout16), jnp.float32),
    )(p, h, wrelK, brelT, wrootK)


def _tc_head(p, h, wrelK, brelT, wrootK, wpack):
    def body(p_ref, h_ref, wr, br, wo, wp, o_ref):
        agg = p_ref[0] + p_ref[1]
        y = (jnp.dot(agg, wr[...], preferred_element_type=jnp.float32, precision=lax.Precision.HIGHEST)
             + br[...]
             + jnp.dot(h_ref[...], wo[...],
                       preferred_element_type=jnp.float32, precision=lax.Precision.HIGHEST))
        h4 = _gelu(y)

        @pl.when(pl.program_id(0) == 0)
        def _init():
            o_ref[...] = jnp.zeros_like(o_ref)

        o_ref[...] += jnp.sum(h4 * wp[...]).reshape(1, 1)

    return pl.pallas_call(
        body,
        grid=(GRID,),
        in_specs=[
            pl.BlockSpec((2, BLK, 512), lambda i: (0, i, 0)),
            pl.BlockSpec((BLK, 512), lambda i: (i, 0)),
            pl.BlockSpec((512, 1024), lambda i: (0, 0)),
            pl.BlockSpec((1, 1024), lambda i: (0, 0)),
            pl.BlockSpec((512, 1024), lambda i: (0, 0)),
            pl.BlockSpec((BLK, 1024), lambda i: (i, 0)),
        ],
        out_specs=pl.BlockSpec((1, 1), lambda i: (0, 0)),
        out_shape=jax.ShapeDtypeStruct((1, 1), jnp.float32),
    )(p, h, wrelK, brelT, wrootK, wpack)


def _bin_membership():
    ar = np.arange(FINAL)
    starts = (ar * N_NODES) // FINAL
    ends = -((-((ar + 1) * N_NODES)) // FINAL)
    p = np.arange(N_NODES)
    m = (p[None, :] >= starts[:, None]) & (p[None, :] < ends[:, None])
    return m.astype(np.float32) / (ends - starts)[:, None].astype(np.float32)


_MPOOL = _bin_membership()


def _head_weights(Wout):
    wv = jnp.dot(Wout.reshape(8 * 8, FINAL), jnp.asarray(_MPOOL),
                 preferred_element_type=jnp.float32, precision=lax.Precision.HIGHEST)
    flat = jnp.concatenate(
        [wv.reshape(-1), jnp.zeros(((N_PAD - N_NODES) * 64,), jnp.float32)])
    return flat.reshape(NP, 16 * 64)


def kernel(x, edge_index, Wrel1, brel1, Wroot1, Wrel2, brel2, Wroot2,
           Wrel3, brel3, Wroot3, Wrel4, brel4, Wroot4, Wout, bout):
    src = edge_index[0]
    dst = edge_index[1]
    npad = E_PAD - src.shape[0]
    src_p = jnp.concatenate([src, jnp.zeros((npad,), jnp.int32)])
    dst_p = jnp.concatenate([dst, jnp.full((npad,), DUMP_ROW, jnp.int32)])
    cfgA, cfgB = _CFG[8], _CFG[32]
    rowsA = 16 * (cfgA[1] + cfgA[3])
    rowsB = 16 * (cfgB[1] + cfgB[3])
    srcA = src_p.reshape(rowsA, cfgA[0])
    dstA = dst_p.reshape(rowsA, cfgA[0])
    srcB = src_p.reshape(rowsB, cfgB[0])
    dstB = dst_p.reshape(rowsB, cfgB[0])

    zeros = {c: jnp.zeros((ROWS_PER_SUB, c), jnp.float32) for c in (8, 16, 32)}
    x2 = jnp.pad(x[:, 0], (0, N_PAD - N_NODES)).reshape(NP, 16)
    wpack = _head_weights(Wout)

    g1 = _tc_premul(x2, _kron16(Wrel1.T))
    p1 = _make_seg_sum(8)(srcA, dstA, g1.reshape(N_PAD, 8), zeros[8])
    h1 = _tc_layer1(p1.reshape(2, NP, 128), x2,
                    jnp.tile(brel1, 16).reshape(1, 128), _kron16(Wroot1.T))
    p2 = _make_seg_sum(8)(srcA, dstA, h1.reshape(N_PAD, 8), zeros[8])
    h2 = _tc_layer(p2.reshape(2, NP, 128), h1, _kron16(Wrel2.T),
                   jnp.tile(brel2, 16).reshape(1, 256), _kron16(Wroot2.T))
    p3 = _make_seg_sum(16)(srcA, dstA, h2.reshape(N_PAD, 16), zeros[16])
    h3 = _tc_layer(p3.reshape(2, NP, 256), h2, _kron16(Wrel3.T),
                   jnp.tile(brel3, 16).reshape(1, 512), _kron16(Wroot3.T))
    p4 = _make_seg_sum(32)(srcB, dstB, h3.reshape(N_PAD, 32), zeros[32])
    acc = _tc_head(p4.reshape(2, NP, 512), h3, _kron16(Wrel4.T),
                   jnp.tile(brel4, 16).reshape(1, 1024), _kron16(Wroot4.T),
                   wpack)
    return acc.reshape(1) + bout


# speedup vs baseline: 18.1890x; 1.0056x over previous
import functools

import jax
import jax.numpy as jnp
import numpy as np
from jax import lax
from jax.experimental import pallas as pl
from jax.experimental.pallas import tpu as pltpu
from jax.experimental.pallas import tpu_sc as plsc

N_NODES = 50000
FINAL = 41
N_PAD = 50048
NP = N_PAD // 16
ROWS_PER_SUB = 3128
N_WORKERS = 32
E_PER_W = 25600
E_PAD = N_WORKERS * E_PER_W
DUMP_ROW = 50000
_CFG = {8: (512, 70, 35, 30, 30),
        16: (512, 70, 35, 30, 30),
        32: (128, 280, 35, 120, 30)}
_FAST_CID = 0
NBUF = 4


def _make_seg_sum(C):
    CHUNK, NCHF, IBCF, NCHS, IBCS = _CFG[C]
    IBC = max(IBCF, IBCS)
    mesh = plsc.VectorSubcoreMesh(core_axis_name="c", subcore_axis_name="s")

    @functools.partial(
        pl.kernel,
        out_type=jax.ShapeDtypeStruct((2, N_PAD, C), jnp.float32),
        mesh=mesh,
        scratch_types=[
            pltpu.VMEM((IBC, CHUNK), jnp.int32),
            pltpu.VMEM((IBC, CHUNK), jnp.int32),
            pltpu.VMEM((CHUNK, C), jnp.float32),
            pltpu.VMEM((CHUNK, C), jnp.float32),
            pltpu.VMEM((CHUNK, C), jnp.float32),
            pltpu.VMEM((CHUNK, C), jnp.float32),
            pltpu.VMEM_SHARED((N_PAD, C), jnp.float32),
            pltpu.SemaphoreType.DMA((NBUF,)),
            pltpu.SemaphoreType.DMA((NBUF,)),
        ],
        compiler_params=pltpu.CompilerParams(use_tc_tiling_on_sc=False),
    )
    def seg_sum(src_hbm, dst_hbm, h_hbm, zeros_hbm, out_hbm,
                sblk, dblk, r0, r1, r2, r3, agg_sh, gsem, ssem):
        rows = [r0, r1, r2, r3]
        cid = lax.axis_index("c")
        sid = lax.axis_index("s")
        wid = cid * 16 + sid
        sl_me = pl.ds(sid * ROWS_PER_SUB, ROWS_PER_SUB)

        def g_start(j, b):
            pltpu.async_copy(h_hbm.at[sblk.at[j]], rows[b], gsem.at[b])

        def g_wait(j, b):
            pltpu.make_async_copy(h_hbm.at[sblk.at[j]], rows[b],
                                  gsem.at[b]).wait()

        def s_start(j, b):
            pltpu.async_copy(rows[b], agg_sh.at[dblk.at[j]], ssem.at[b],
                             add=True)

        def s_wait(j, b):
            pltpu.make_async_copy(rows[b], agg_sh.at[dblk.at[j]],
                                  ssem.at[b]).wait()

        pltpu.sync_copy(zeros_hbm.at[sl_me], agg_sh.at[sl_me])
        plsc.subcore_barrier()

        def run_block(blk0, ibc):
            pltpu.sync_copy(src_hbm.at[pl.ds(blk0, ibc)], sblk.at[pl.ds(0, ibc)])
            pltpu.sync_copy(dst_hbm.at[pl.ds(blk0, ibc)], dblk.at[pl.ds(0, ibc)])
            for b in range(NBUF):
                g_start(b, b)
            for j in range(2):
                g_wait(j, j)
                s_start(j, j)

            n_grp = (ibc - 6) // NBUF

            def grp(i, carry):
                j0 = 2 + i * NBUF
                for q in range(NBUF):
                    j = j0 + q
                    b = (2 + q) % NBUF
                    g_wait(j, b)
                    s_start(j, b)
                    b2 = q
                    s_wait(j - 2, b2)
                    g_start(j + 2, b2)
                return carry

            lax.fori_loop(0, n_grp, grp, 0)

            for j in range(2 + n_grp * NBUF, ibc):
                b = j % NBUF
                g_wait(j, b)
                s_start(j, b)
                if j + 2 < ibc:
                    s_wait(j - 2, (j + 2) % NBUF)
                    g_start(j + 2, (j + 2) % NBUF)
            for b in range(NBUF):
                s_wait(0, b)

        fast_row0 = 0 if _FAST_CID == 0 else 16 * NCHS
        slow_row0 = 16 * NCHF if _FAST_CID == 0 else 0

        @pl.when(cid == _FAST_CID)
        def _fast_side():
            for t in range(NCHF // IBCF):
                run_block(fast_row0 + sid * NCHF + t * IBCF, IBCF)

        @pl.when(cid != _FAST_CID)
        def _slow_side():
            for t in range(NCHS // IBCS):
                run_block(slow_row0 + sid * NCHS + t * IBCS, IBCS)

        plsc.subcore_barrier()

        pltpu.sync_copy(agg_sh.at[sl_me], out_hbm.at[cid, sl_me])

    return seg_sum


def _gelu(y):
    return 0.5 * y * (1.0 + lax.erf(y * np.float32(1.0 / np.sqrt(2.0))))


def _kron16(w):
    return jnp.kron(jnp.eye(16, dtype=jnp.float32), w)


BLK = 184
GRID = NP // BLK


def _tc_premul(x2, wx):
    def body(x_ref, w_ref, o_ref):
        o_ref[...] = jnp.dot(x_ref[...], w_ref[...],
                             preferred_element_type=jnp.float32, precision=lax.Precision.HIGHEST)

    return pl.pallas_call(
        body,
        grid=(GRID,),
        in_specs=[
            pl.BlockSpec((BLK, 16), lambda i: (i, 0)),
            pl.BlockSpec((16, 128), lambda i: (0, 0)),
        ],
        out_specs=pl.BlockSpec((BLK, 128), lambda i: (i, 0)),
        out_shape=jax.ShapeDtypeStruct((NP, 128), jnp.float32),
    )(x2, wx)


def _tc_layer1(p, x2, brelT, wrootx):
    def body(p_ref, x_ref, br, wo, o_ref):
        y = (p_ref[0] + p_ref[1] + br[...]
             + jnp.dot(x_ref[...], wo[...],
                       preferred_element_type=jnp.float32, precision=lax.Precision.HIGHEST))
        o_ref[...] = _gelu(y)

    return pl.pallas_call(
        body,
        grid=(GRID,),
        in_specs=[
            pl.BlockSpec((2, BLK, 128), lambda i: (0, i, 0)),
            pl.BlockSpec((BLK, 16), lambda i: (i, 0)),
            pl.BlockSpec((1, 128), lambda i: (0, 0)),
            pl.BlockSpec((16, 128), lambda i: (0, 0)),
        ],
        out_specs=pl.BlockSpec((BLK, 128), lambda i: (i, 0)),
        out_shape=jax.ShapeDtypeStruct((NP, 128), jnp.float32),
    )(p, x2, brelT, wrootx)


def _tc_layer(p, h, wrelK, brelT, wrootK):
    Cin16, Cout16 = wrelK.shape

    def body(p_ref, h_ref, wr, br, wo, o_ref):
        agg = p_ref[0] + p_ref[1]
        y = (jnp.dot(agg, wr[...], preferred_element_type=jnp.float32, precision=lax.Precision.HIGHEST)
             + br[...]
             + jnp.dot(h_ref[...], wo[...],
                       preferred_element_type=jnp.float32, precision=lax.Precision.HIGHEST))
        o_ref[...] = _gelu(y)

    return pl.pallas_call(
        body,
        grid=(GRID,),
        in_specs=[
            pl.BlockSpec((2, BLK, Cin16), lambda i: (0, i, 0)),
            pl.BlockSpec((BLK, Cin16), lambda i: (i, 0)),
            pl.BlockSpec((Cin16, Cout16), lambda i: (0, 0)),
            pl.BlockSpec((1, Cout16), lambda i: (0, 0)),
            pl.BlockSpec((Cin16, Cout16), lambda i: (0, 0)),
        ],
        out_specs=pl.BlockSpec((BLK, Cout16), lambda i: (i, 0)),
        out_shape=jax.ShapeDtypeStruct((NP, Cout16), jnp.float32),
    )(p, h, wrelK, brelT, wrootK)


def _tc_head(p, h, wrelK, brelT, wrootK, wpack):
    def body(p_ref, h_ref, wr, br, wo, wp, o_ref):
        agg = p_ref[0] + p_ref[1]
        y = (jnp.dot(agg, wr[...], preferred_element_type=jnp.float32, precision=lax.Precision.HIGHEST)
             + br[...]
             + jnp.dot(h_ref[...], wo[...],
                       preferred_element_type=jnp.float32, precision=lax.Precision.HIGHEST))
        h4 = _gelu(y)

        @pl.when(pl.program_id(0) == 0)
        def _init():
            o_ref[...] = jnp.zeros_like(o_ref)

        o_ref[...] += jnp.sum(h4 * wp[...]).reshape(1, 1)

    return pl.pallas_call(
        body,
        grid=(GRID,),
        in_specs=[
            pl.BlockSpec((2, BLK, 512), lambda i: (0, i, 0)),
            pl.BlockSpec((BLK, 512), lambda i: (i, 0)),
            pl.BlockSpec((512, 1024), lambda i: (0, 0)),
            pl.BlockSpec((1, 1024), lambda i: (0, 0)),
            pl.BlockSpec((512, 1024), lambda i: (0, 0)),
            pl.BlockSpec((BLK, 1024), lambda i: (i, 0)),
        ],
        out_specs=pl.BlockSpec((1, 1), lambda i: (0, 0)),
        out_shape=jax.ShapeDtypeStruct((1, 1), jnp.float32),
    )(p, h, wrelK, brelT, wrootK, wpack)


def _bin_membership():
    ar = np.arange(FINAL)
    starts = (ar * N_NODES) // FINAL
    ends = -((-((ar + 1) * N_NODES)) // FINAL)
    p = np.arange(N_NODES)
    m = (p[None, :] >= starts[:, None]) & (p[None, :] < ends[:, None])
    return m.astype(np.float32) / (ends - starts)[:, None].astype(np.float32)


_MPOOL = _bin_membership()


def _head_weights(Wout):
    wv = jnp.dot(Wout.reshape(8 * 8, FINAL), jnp.asarray(_MPOOL),
                 preferred_element_type=jnp.float32, precision=lax.Precision.HIGHEST)
    flat = jnp.concatenate(
        [wv.reshape(-1), jnp.zeros(((N_PAD - N_NODES) * 64,), jnp.float32)])
    return flat.reshape(NP, 16 * 64)


def kernel(x, edge_index, Wrel1, brel1, Wroot1, Wrel2, brel2, Wroot2,
           Wrel3, brel3, Wroot3, Wrel4, brel4, Wroot4, Wout, bout):
    src = edge_index[0]
    dst = edge_index[1]
    npad = E_PAD - src.shape[0]
    src_p = jnp.concatenate([src, jnp.zeros((npad,), jnp.int32)])
    dst_p = jnp.concatenate([dst, jnp.full((npad,), DUMP_ROW, jnp.int32)])
    cfgA, cfgB = _CFG[8], _CFG[32]
    rowsA = 16 * (cfgA[1] + cfgA[3])
    rowsB = 16 * (cfgB[1] + cfgB[3])
    srcA = src_p.reshape(rowsA, cfgA[0])
    dstA = dst_p.reshape(rowsA, cfgA[0])
    srcB = src_p.reshape(rowsB, cfgB[0])
    dstB = dst_p.reshape(rowsB, cfgB[0])

    zeros = {c: jnp.zeros((N_PAD, c), jnp.float32) for c in (8, 16, 32)}
    x2 = jnp.pad(x[:, 0], (0, N_PAD - N_NODES)).reshape(NP, 16)
    wpack = _head_weights(Wout)

    g1 = _tc_premul(x2, _kron16(Wrel1.T))
    p1 = _make_seg_sum(8)(srcA, dstA, g1.reshape(N_PAD, 8), zeros[8])
    h1 = _tc_layer1(p1.reshape(2, NP, 128), x2,
                    jnp.tile(brel1, 16).reshape(1, 128), _kron16(Wroot1.T))
    p2 = _make_seg_sum(8)(srcA, dstA, h1.reshape(N_PAD, 8), zeros[8])
    h2 = _tc_layer(p2.reshape(2, NP, 128), h1, _kron16(Wrel2.T),
                   jnp.tile(brel2, 16).reshape(1, 256), _kron16(Wroot2.T))
    p3 = _make_seg_sum(16)(srcA, dstA, h2.reshape(N_PAD, 16), zeros[16])
    h3 = _tc_layer(p3.reshape(2, NP, 256), h2, _kron16(Wrel3.T),
                   jnp.tile(brel3, 16).reshape(1, 512), _kron16(Wroot3.T))
    p4 = _make_seg_sum(32)(srcB, dstB, h3.reshape(N_PAD, 32), zeros[32])
    acc = _tc_head(p4.reshape(2, NP, 512), h3, _kron16(Wrel4.T),
                   jnp.tile(brel4, 16).reshape(1, 1024), _kron16(Wroot4.T),
                   wpack)
    return acc.reshape(1) + bout
